# baseline (speedup 1.0000x reference)
import jax
import jax.numpy as jnp
from jax.experimental import pallas as pl
from jax.experimental.pallas import tpu as pltpu

_ROWS = 16384
_COLS = 4096
_PERIOD = 4096
_N_CHUNKS = 8
_CHUNK_ROWS = _ROWS // _N_CHUNKS


def _copy_body(in_ref, out_ref, sem):
    copies = [
        pltpu.make_async_copy(
            in_ref.at[pl.ds(i * _CHUNK_ROWS, _CHUNK_ROWS)],
            out_ref.at[pl.ds(i * _CHUNK_ROWS, _CHUNK_ROWS)],
            sem,
        )
        for i in range(_N_CHUNKS)
    ]
    for c in copies:
        c.start()
    for c in copies:
        c.wait()


def kernel(x):
    out = pl.pallas_call(
        _copy_body,
        in_specs=[pl.BlockSpec(memory_space=pltpu.MemorySpace.HBM)],
        out_specs=pl.BlockSpec(memory_space=pltpu.MemorySpace.HBM),
        out_shape=jax.ShapeDtypeStruct((_ROWS, _COLS), jnp.float32),
        scratch_shapes=[pltpu.SemaphoreType.DMA],
    )(x)
    return jnp.reshape(out, (_ROWS // _PERIOD, _PERIOD, _COLS))


# speedup vs baseline: 40.5999x; 40.5999x over previous
import jax
import jax.numpy as jnp
from jax.experimental import pallas as pl
from jax.experimental.pallas import tpu as pltpu

_ROWS = 16384
_COLS = 4096
_PERIOD = 4096
_N_CHUNKS = 32
_CHUNK_ROWS = _ROWS // _N_CHUNKS
_N_BUF = 4


def _copy_body(in_hbm, out_hbm, buf, in_sem, out_sem):
    def in_copy(i, slot):
        return pltpu.make_async_copy(
            in_hbm.at[pl.ds(i * _CHUNK_ROWS, _CHUNK_ROWS)],
            buf.at[slot],
            in_sem.at[slot],
        )

    def out_copy(i, slot):
        return pltpu.make_async_copy(
            buf.at[slot],
            out_hbm.at[pl.ds(i * _CHUNK_ROWS, _CHUNK_ROWS)],
            out_sem.at[slot],
        )

    for s in range(min(_N_BUF, _N_CHUNKS)):
        in_copy(s, s).start()
    for i in range(_N_CHUNKS):
        slot = i % _N_BUF
        in_copy(i, slot).wait()
        out_copy(i, slot).start()
        oldest = i - (_N_BUF - 1)
        nxt = i + 1
        if oldest >= 0 and nxt < _N_CHUNKS:
            out_copy(oldest, oldest % _N_BUF).wait()
            in_copy(nxt, nxt % _N_BUF).start()
    for i in range(max(_N_CHUNKS - _N_BUF, 0), _N_CHUNKS):
        out_copy(i, i % _N_BUF).wait()


def kernel(x):
    out = pl.pallas_call(
        _copy_body,
        in_specs=[pl.BlockSpec(memory_space=pltpu.MemorySpace.HBM)],
        out_specs=pl.BlockSpec(memory_space=pltpu.MemorySpace.HBM),
        out_shape=jax.ShapeDtypeStruct((_ROWS, _COLS), jnp.float32),
        scratch_shapes=[
            pltpu.VMEM((_N_BUF, _CHUNK_ROWS, _COLS), jnp.float32),
            pltpu.SemaphoreType.DMA((_N_BUF,)),
            pltpu.SemaphoreType.DMA((_N_BUF,)),
        ],
    )(x)
    return jnp.reshape(out, (_ROWS // _PERIOD, _PERIOD, _COLS))


# speedup vs baseline: 48.4717x; 1.1939x over previous
import jax
import jax.numpy as jnp
from jax.experimental import pallas as pl

_ROWS = 16384
_COLS = 4096
_PERIOD = 4096
_BLOCK_ROWS = 256


def _copy_body(in_ref, out_ref):
    out_ref[...] = in_ref[...]


def kernel(x):
    grid = (_ROWS // _BLOCK_ROWS,)
    out = pl.pallas_call(
        _copy_body,
        grid=grid,
        in_specs=[pl.BlockSpec((_BLOCK_ROWS, _COLS), lambda i: (i, 0))],
        out_specs=pl.BlockSpec((_BLOCK_ROWS, _COLS), lambda i: (i, 0)),
        out_shape=jax.ShapeDtypeStruct((_ROWS, _COLS), jnp.float32),
    )(x)
    return jnp.reshape(out, (_ROWS // _PERIOD, _PERIOD, _COLS))
